# Initial kernel scaffold; baseline (speedup 1.0000x reference)
import jax
import jax.numpy as jnp
from jax.experimental import pallas as pl


def kernel(x, edge_index, W1, b1, W2, b2):
    raise NotImplementedError("write your pallas kernel here")


# speedup vs baseline: 21.9999x; 21.9999x over previous
import functools

import jax
import jax.numpy as jnp
from jax import lax
from jax.experimental import pallas as pl
from jax.experimental.pallas import tpu as pltpu
from jax.experimental.pallas import tpu_sc as plsc

N = 10000
E = 320000
DIN = 128
HID = 16
NCLS = 40

NCORE = 2
NSUB = 16
NW = NCORE * NSUB
B = 128
K = 80
E_PAD = NW * B * K
NPAD = 10112
RPS = NPAD // NSUB
DEGW = 16
RB = 1000


def _sc_agg(D):
    mesh = plsc.VectorSubcoreMesh(core_axis_name="c", subcore_axis_name="s")

    @functools.partial(
        pl.kernel,
        out_type=jax.ShapeDtypeStruct((NCORE, NPAD, D), jnp.float32),
        mesh=mesh,
        scratch_types=[
            pltpu.VMEM((K, B), jnp.int32),
            pltpu.VMEM((K, B), jnp.int32),
            pltpu.VMEM((B, D), jnp.float32),
            pltpu.VMEM_SHARED((NPAD, D), jnp.float32),
            pltpu.SemaphoreType.DMA,
        ],
        compiler_params=pltpu.CompilerParams(use_tc_tiling_on_sc=False),
    )
    def agg(hs_hbm, src_hbm, dst_hbm, zeros_hbm, out_hbm,
            src_v, dst_v, rows_v, acc_sh, sem):
        c = lax.axis_index("c")
        s = lax.axis_index("s")
        w = s * NCORE + c
        pltpu.sync_copy(src_hbm.at[pl.ds(w * K, K)], src_v)
        pltpu.sync_copy(dst_hbm.at[pl.ds(w * K, K)], dst_v)
        pltpu.sync_copy(zeros_hbm, acc_sh.at[pl.ds(s * RPS, RPS)])
        plsc.subcore_barrier()

        def step(j, carry):
            pltpu.async_copy(hs_hbm.at[src_v.at[j]], rows_v, sem).wait()
            pltpu.sync_copy(rows_v, acc_sh.at[dst_v.at[j]], add=True)
            return carry

        lax.fori_loop(0, K, step, 0)
        plsc.subcore_barrier()
        pltpu.sync_copy(acc_sh.at[pl.ds(s * RPS, RPS)],
                        out_hbm.at[c, pl.ds(s * RPS, RPS)])

    return agg


def _sc_deg():
    mesh = plsc.VectorSubcoreMesh(core_axis_name="c", subcore_axis_name="s")

    @functools.partial(
        pl.kernel,
        out_type=jax.ShapeDtypeStruct((NCORE, NPAD, DEGW), jnp.float32),
        mesh=mesh,
        scratch_types=[
            pltpu.VMEM((K, B), jnp.int32),
            pltpu.VMEM((B, DEGW), jnp.float32),
            pltpu.VMEM_SHARED((NPAD, DEGW), jnp.float32),
        ],
        compiler_params=pltpu.CompilerParams(use_tc_tiling_on_sc=False),
    )
    def deg(dst_hbm, ones_hbm, zeros_hbm, out_hbm, dst_v, ones_v, acc_sh):
        c = lax.axis_index("c")
        s = lax.axis_index("s")
        w = s * NCORE + c
        pltpu.sync_copy(dst_hbm.at[pl.ds(w * K, K)], dst_v)
        pltpu.sync_copy(ones_hbm, ones_v)
        pltpu.sync_copy(zeros_hbm, acc_sh.at[pl.ds(s * RPS, RPS)])
        plsc.subcore_barrier()

        def step(j, carry):
            pltpu.sync_copy(ones_v, acc_sh.at[dst_v.at[j]], add=True)
            return carry

        lax.fori_loop(0, K, step, 0)
        plsc.subcore_barrier()
        pltpu.sync_copy(acc_sh.at[pl.ds(s * RPS, RPS)],
                        out_hbm.at[c, pl.ds(s * RPS, RPS)])

    return deg


def _tc1(x, W1, degp):
    def body(x_ref, w_ref, d_ref, h_ref, hs_ref, dinv_ref):
        h = jnp.dot(x_ref[...], w_ref[...], preferred_element_type=jnp.float32)
        deg = d_ref[0, :, :1] + d_ref[1, :, :1] + 1.0
        dinv = lax.rsqrt(deg)
        h_ref[...] = h
        hs_ref[...] = h * dinv
        dinv_ref[...] = dinv

    return pl.pallas_call(
        body,
        grid=(N // RB,),
        in_specs=[
            pl.BlockSpec((RB, DIN), lambda i: (i, 0)),
            pl.BlockSpec((DIN, HID), lambda i: (0, 0)),
            pl.BlockSpec((NCORE, RB, DEGW), lambda i: (0, i, 0)),
        ],
        out_specs=[
            pl.BlockSpec((RB, HID), lambda i: (i, 0)),
            pl.BlockSpec((RB, HID), lambda i: (i, 0)),
            pl.BlockSpec((RB, 1), lambda i: (i, 0)),
        ],
        out_shape=[
            jax.ShapeDtypeStruct((N, HID), jnp.float32),
            jax.ShapeDtypeStruct((N, HID), jnp.float32),
            jax.ShapeDtypeStruct((N, 1), jnp.float32),
        ],
    )(x, W1, degp)


def _tc2(p, h1, dinv, b1, W2):
    def body(p_ref, h1_ref, dinv_ref, b1_ref, w2_ref, h2_ref, hs2_ref):
        dinv = dinv_ref[...]
        agg = p_ref[0] + p_ref[1]
        z1 = dinv * agg + (dinv * dinv) * h1_ref[...] + b1_ref[...]
        z1 = jnp.maximum(z1, 0.0)
        h2 = jnp.dot(z1, w2_ref[...], preferred_element_type=jnp.float32)
        h2_ref[...] = h2
        hs2_ref[...] = h2 * dinv

    return pl.pallas_call(
        body,
        grid=(N // RB,),
        in_specs=[
            pl.BlockSpec((NCORE, RB, HID), lambda i: (0, i, 0)),
            pl.BlockSpec((RB, HID), lambda i: (i, 0)),
            pl.BlockSpec((RB, 1), lambda i: (i, 0)),
            pl.BlockSpec((1, HID), lambda i: (0, 0)),
            pl.BlockSpec((HID, NCLS), lambda i: (0, 0)),
        ],
        out_specs=[
            pl.BlockSpec((RB, NCLS), lambda i: (i, 0)),
            pl.BlockSpec((RB, NCLS), lambda i: (i, 0)),
        ],
        out_shape=[
            jax.ShapeDtypeStruct((N, NCLS), jnp.float32),
            jax.ShapeDtypeStruct((N, NCLS), jnp.float32),
        ],
    )(p, h1, dinv, b1.reshape(1, HID), W2)


def _tc3(q, h2, dinv, b2):
    def body(q_ref, h2_ref, dinv_ref, b2_ref, out_ref):
        dinv = dinv_ref[...]
        agg = q_ref[0] + q_ref[1]
        out_ref[...] = (dinv * agg + (dinv * dinv) * h2_ref[...]
                        + b2_ref[...])

    return pl.pallas_call(
        body,
        grid=(N // RB,),
        in_specs=[
            pl.BlockSpec((NCORE, RB, NCLS), lambda i: (0, i, 0)),
            pl.BlockSpec((RB, NCLS), lambda i: (i, 0)),
            pl.BlockSpec((RB, 1), lambda i: (i, 0)),
            pl.BlockSpec((1, NCLS), lambda i: (0, 0)),
        ],
        out_specs=pl.BlockSpec((RB, NCLS), lambda i: (i, 0)),
        out_shape=jax.ShapeDtypeStruct((N, NCLS), jnp.float32),
    )(q, h2, dinv, b2.reshape(1, NCLS))


def kernel(x, edge_index, W1, b1, W2, b2):
    src = edge_index[0]
    dst = edge_index[1]
    pad = E_PAD - E
    src_p = jnp.concatenate(
        [src, jnp.zeros((pad,), jnp.int32)]).reshape(NW * K, B)
    dst_p = jnp.concatenate(
        [dst, jnp.full((pad,), N, jnp.int32)]).reshape(NW * K, B)

    ones_deg = jnp.ones((B, DEGW), jnp.float32)
    zeros_deg = jnp.zeros((RPS, DEGW), jnp.float32)
    zeros_h = jnp.zeros((RPS, HID), jnp.float32)
    zeros_c = jnp.zeros((RPS, NCLS), jnp.float32)

    degp = _sc_deg()(dst_p, ones_deg, zeros_deg)
    h1, hs1, dinv = _tc1(x, W1, degp)
    p = _sc_agg(HID)(hs1, src_p, dst_p, zeros_h)
    h2, hs2 = _tc2(p, h1, dinv, b1, W2)
    q = _sc_agg(NCLS)(hs2, src_p, dst_p, zeros_c)
    return _tc3(q, h2, dinv, b2)


# speedup vs baseline: 28.1998x; 1.2818x over previous
import functools

import jax
import jax.numpy as jnp
from jax import lax
from jax.experimental import pallas as pl
from jax.experimental.pallas import tpu as pltpu
from jax.experimental.pallas import tpu_sc as plsc

N = 10000
E = 320000
DIN = 128
HID = 16
NCLS = 40

NCORE = 2
NSUB = 16
NW = NCORE * NSUB
B = 128
K = 80
E_PAD = NW * B * K
NPAD = 10112
RPS = NPAD // NSUB
DEGW = 16
NB = 4

_SC_PARAMS = pltpu.CompilerParams(use_tc_tiling_on_sc=False)


def _sc_agg():
    D = HID
    mesh = plsc.VectorSubcoreMesh(core_axis_name="c", subcore_axis_name="s")

    @functools.partial(
        pl.kernel,
        out_type=jax.ShapeDtypeStruct((NCORE, NPAD, D), jnp.float32),
        mesh=mesh,
        scratch_types=[
            pltpu.VMEM((K, B), jnp.int32),
            pltpu.VMEM((K, B), jnp.int32),
            pltpu.VMEM((NB, B, D), jnp.float32),
            pltpu.VMEM_SHARED((NPAD, D), jnp.float32),
        ] + [pltpu.SemaphoreType.DMA] * (2 * NB),
        compiler_params=_SC_PARAMS,
    )
    def agg(hs_hbm, src_hbm, dst_hbm, zeros_hbm, out_hbm,
            src_v, dst_v, rows_v, acc_sh, *sems):
        gsem = sems[:NB]
        ssem = sems[NB:]
        c = lax.axis_index("c")
        s = lax.axis_index("s")
        w = s * NCORE + c
        pltpu.sync_copy(src_hbm.at[pl.ds(w * K, K)], src_v)
        pltpu.sync_copy(dst_hbm.at[pl.ds(w * K, K)], dst_v)
        pltpu.sync_copy(zeros_hbm, acc_sh.at[pl.ds(s * RPS, RPS)])
        plsc.subcore_barrier()

        pltpu.async_copy(hs_hbm.at[src_v.at[0]], rows_v.at[0], gsem[0])

        def outer(t, carry):
            for b in range(NB):
                j = t * NB + b
                bn = (b + 1) % NB
                jn = j + 1
                pltpu.make_async_copy(
                    hs_hbm.at[src_v.at[j]], rows_v.at[b], gsem[b]).wait()
                pltpu.async_copy(rows_v.at[b],
                                 acc_sh.at[dst_v.at[j]], ssem[b], add=True)

                @pl.when(jnp.logical_and(jn >= NB, jn < K))
                def _():
                    pltpu.make_async_copy(
                        rows_v.at[bn],
                        acc_sh.at[dst_v.at[jn - NB]], ssem[bn]).wait()

                @pl.when(jn < K)
                def _():
                    pltpu.async_copy(
                        hs_hbm.at[src_v.at[jn]], rows_v.at[bn], gsem[bn])
            return carry

        lax.fori_loop(0, K // NB, outer, 0)
        for cch in range(K - NB, K):
            pltpu.make_async_copy(
                rows_v.at[cch % NB],
                acc_sh.at[dst_v.at[cch]], ssem[cch % NB]).wait()
        plsc.subcore_barrier()
        pltpu.sync_copy(acc_sh.at[pl.ds(s * RPS, RPS)],
                        out_hbm.at[c, pl.ds(s * RPS, RPS)])

    return agg


def _sc_deg():
    mesh = plsc.VectorSubcoreMesh(core_axis_name="c", subcore_axis_name="s")

    @functools.partial(
        pl.kernel,
        out_type=jax.ShapeDtypeStruct((NCORE, NPAD, DEGW), jnp.float32),
        mesh=mesh,
        scratch_types=[
            pltpu.VMEM((K, B), jnp.int32),
            pltpu.VMEM((B, DEGW), jnp.float32),
            pltpu.VMEM_SHARED((NPAD, DEGW), jnp.float32),
        ],
        compiler_params=_SC_PARAMS,
    )
    def deg(dst_hbm, ones_hbm, zeros_hbm, out_hbm, dst_v, ones_v, acc_sh):
        c = lax.axis_index("c")
        s = lax.axis_index("s")
        w = s * NCORE + c
        pltpu.sync_copy(dst_hbm.at[pl.ds(w * K, K)], dst_v)
        pltpu.sync_copy(ones_hbm, ones_v)
        pltpu.sync_copy(zeros_hbm, acc_sh.at[pl.ds(s * RPS, RPS)])
        plsc.subcore_barrier()

        def step(j, carry):
            pltpu.sync_copy(ones_v, acc_sh.at[dst_v.at[j]], add=True)
            return carry

        lax.fori_loop(0, K, step, 0)
        plsc.subcore_barrier()
        pltpu.sync_copy(acc_sh.at[pl.ds(s * RPS, RPS)],
                        out_hbm.at[c, pl.ds(s * RPS, RPS)])

    return deg


def _tc_mm1(x, W1):
    def body(x_ref, w_ref, h_ref):
        h_ref[...] = jnp.dot(x_ref[...], w_ref[...],
                             preferred_element_type=jnp.float32)

    return pl.pallas_call(
        body,
        out_shape=jax.ShapeDtypeStruct((N, HID), jnp.float32),
    )(x, W1)


def _tc_scale1(degp, h1):
    def body(d_ref, h_ref, hs_ref, dinv_ref):
        deg = d_ref[0, :, :1] + d_ref[1, :, :1] + 1.0
        dinv = lax.rsqrt(deg)
        hs_ref[...] = h_ref[...] * dinv
        dinv_ref[...] = dinv

    return pl.pallas_call(
        body,
        out_shape=[
            jax.ShapeDtypeStruct((N, HID), jnp.float32),
            jax.ShapeDtypeStruct((N, 1), jnp.float32),
        ],
    )(degp, h1)


def _tc_relu(p, h1, dinv, b1):
    def body(p_ref, h1_ref, dinv_ref, b1_ref, z_ref, zs_ref):
        dinv = dinv_ref[...]
        z = dinv * (p_ref[0] + p_ref[1]) + (dinv * dinv) * h1_ref[...] \
            + b1_ref[...]
        z = jnp.maximum(z, 0.0)
        z_ref[...] = z
        zs_ref[...] = z * dinv

    return pl.pallas_call(
        body,
        out_shape=[
            jax.ShapeDtypeStruct((N, HID), jnp.float32),
            jax.ShapeDtypeStruct((N, HID), jnp.float32),
        ],
    )(p, h1, dinv, b1.reshape(1, HID))


def _tc_out(q, z1, dinv, W2, b2):
    def body(q_ref, z_ref, dinv_ref, w2_ref, b2_ref, out_ref):
        dinv = dinv_ref[...]
        agg = dinv * (q_ref[0] + q_ref[1]) + (dinv * dinv) * z_ref[...]
        out_ref[...] = jnp.dot(agg, w2_ref[...],
                               preferred_element_type=jnp.float32) \
            + b2_ref[...]

    return pl.pallas_call(
        body,
        out_shape=jax.ShapeDtypeStruct((N, NCLS), jnp.float32),
    )(q, z1, dinv, W2, b2.reshape(1, NCLS))


def kernel(x, edge_index, W1, b1, W2, b2):
    src = edge_index[0]
    dst = edge_index[1]
    pad = E_PAD - E
    src_p = jnp.concatenate(
        [src, jnp.zeros((pad,), jnp.int32)]).reshape(NW * K, B)
    dst_p = jnp.concatenate(
        [dst, jnp.full((pad,), N, jnp.int32)]).reshape(NW * K, B)

    ones_deg = jnp.ones((B, DEGW), jnp.float32)
    zeros_deg = jnp.zeros((RPS, DEGW), jnp.float32)
    zeros_h = jnp.zeros((RPS, HID), jnp.float32)

    agg = _sc_agg()
    degp = _sc_deg()(dst_p, ones_deg, zeros_deg)
    h1 = _tc_mm1(x, W1)
    hs1, dinv = _tc_scale1(degp[:, :N], h1)
    p = agg(hs1, src_p, dst_p, zeros_h)
    z1, zs1 = _tc_relu(p[:, :N], h1, dinv, b1)
    q = agg(zs1, src_p, dst_p, zeros_h)
    return _tc_out(q[:, :N], z1, dinv, W2, b2)
